# Initial kernel scaffold; baseline (speedup 1.0000x reference)
import jax
import jax.numpy as jnp
from jax.experimental import pallas as pl


def kernel(field, indices):
    raise NotImplementedError("write your pallas kernel here")


# speedup vs baseline: 2.9235x; 2.9235x over previous
import jax
import jax.numpy as jnp
from jax.experimental import pallas as pl
from jax.experimental.pallas import tpu as pltpu


def _copy_body(idx_ref, in_ref, out_ref):
    out_ref[...] = in_ref[...]


def kernel(field, indices):
    levels, lat, lon = field.shape
    n = indices.shape[0]
    grid_spec = pltpu.PrefetchScalarGridSpec(
        num_scalar_prefetch=1,
        grid=(n,),
        in_specs=[pl.BlockSpec((1, lat, lon), lambda i, idx: (idx[i], 0, 0))],
        out_specs=pl.BlockSpec((1, lat, lon), lambda i, idx: (i, 0, 0)),
    )
    return pl.pallas_call(
        _copy_body,
        grid_spec=grid_spec,
        out_shape=jax.ShapeDtypeStruct((n, lat, lon), field.dtype),
    )(indices, field)
